# baseline (speedup 1.0000x reference)
import jax
import jax.numpy as jnp
from jax.experimental import pallas as pl

B = 8
NN = 32
D = 5
C = 32
TOT = C * C * C


QN = 4
NQ = NN // QN


def _weights_body(dps_ref, ct0_ref, ct1_ref, ct2_ref, w_ref):
    b = pl.program_id(0)
    q = pl.program_id(1)

    def cp(v):
        d = dps_ref[v, 0]
        lo = d[:, :, 0]
        hi = d[:, :, 1]
        diff = hi - lo
        c_iota = jax.lax.broadcasted_iota(jnp.int32, (NQ, C), 1)
        acc = jnp.zeros((NQ, C), jnp.float32)
        for dd in range(D):
            bit = ((c_iota >> (D - 1 - dd)) & 1).astype(jnp.float32)
            acc = acc + lo[dd][:, None] + bit * diff[dd][:, None]
        return acc

    cp0 = cp(0)
    cp1 = cp(1)
    cp2 = cp(2)
    p12 = cp1[:, None, :, None] + cp2[:, None, None, :]
    w_ref[0] = cp0[:, :, None, None] + p12

    @pl.when((b == 0) & (q == 0))
    def _():
        t = jax.lax.broadcasted_iota(jnp.int32, (D, TOT), 1)
        d = jax.lax.broadcasted_iota(jnp.int32, (D, TOT), 0)
        ct0_ref[...] = (t >> (14 - d)) & 1
        ct1_ref[...] = (t >> (9 - d)) & 1
        ct2_ref[...] = (t >> (4 - d)) & 1


def kernel(dp0, dp1, dp2):
    dps = jnp.stack([dp0, dp1, dp2])
    dp_spec = pl.BlockSpec((3, 1, D, NQ, 2), lambda b, q: (0, b, 0, q, 0))
    ct_spec = pl.BlockSpec((D, TOT), lambda b, q: (0, 0))
    out = pl.pallas_call(
        _weights_body,
        grid=(B, QN),
        in_specs=[dp_spec],
        out_specs=[
            ct_spec,
            ct_spec,
            ct_spec,
            pl.BlockSpec((1, NQ, C, C, C), lambda b, q: (b, q, 0, 0, 0)),
        ],
        out_shape=[
            jax.ShapeDtypeStruct((D, TOT), jnp.int32),
            jax.ShapeDtypeStruct((D, TOT), jnp.int32),
            jax.ShapeDtypeStruct((D, TOT), jnp.int32),
            jax.ShapeDtypeStruct((B, NN, C, C, C), jnp.float32),
        ],
    )(dps)
    return tuple(out)


# speedup vs baseline: 1.0963x; 1.0963x over previous
import jax
import jax.numpy as jnp
from jax.experimental import pallas as pl

B = 8
NN = 32
D = 5
C = 32
TOT = C * C * C


QN = 2
NQ = NN // QN


def _weights_body(dps_ref, ct0_ref, ct1_ref, ct2_ref, w_ref):
    b = pl.program_id(0)
    q = pl.program_id(1)

    blk = dps_ref[0]

    def cp(v):
        c_iota = jax.lax.broadcasted_iota(jnp.int32, (NN, C), 1)
        acc = jnp.zeros((NN, C), jnp.float32)
        for dd in range(D):
            lo = blk[v * 2 * D + 2 * dd]
            hi = blk[v * 2 * D + 2 * dd + 1]
            diff = hi - lo
            bit = ((c_iota >> (D - 1 - dd)) & 1).astype(jnp.float32)
            acc = acc + lo[:, None] + bit * diff[:, None]
        return jnp.where(q == 0, acc[:NQ], acc[NQ:])

    cp0 = cp(0)
    cp1 = cp(1)
    cp2 = cp(2)
    p12 = cp1[:, None, :, None] + cp2[:, None, None, :]
    w_ref[0] = cp0[:, :, None, None] + p12

    @pl.when((b == 0) & (q == 0))
    def _():
        t = jax.lax.broadcasted_iota(jnp.int32, (D, TOT), 1)
        d = jax.lax.broadcasted_iota(jnp.int32, (D, TOT), 0)
        ct0_ref[...] = (t >> (14 - d)) & 1
        ct1_ref[...] = (t >> (9 - d)) & 1
        ct2_ref[...] = (t >> (4 - d)) & 1


def kernel(dp0, dp1, dp2):
    dps = jnp.stack([dp0, dp1, dp2], axis=1)
    dps = dps.transpose(0, 1, 2, 4, 3).reshape(B, 3 * D * 2, NN)
    dp_spec = pl.BlockSpec((1, 3 * D * 2, NN), lambda b, q: (b, 0, 0))
    ct_spec = pl.BlockSpec((D, TOT), lambda b, q: (0, 0))
    out = pl.pallas_call(
        _weights_body,
        grid=(B, QN),
        in_specs=[dp_spec],
        out_specs=[
            ct_spec,
            ct_spec,
            ct_spec,
            pl.BlockSpec((1, NQ, C, C, C), lambda b, q: (b, q, 0, 0, 0)),
        ],
        out_shape=[
            jax.ShapeDtypeStruct((D, TOT), jnp.int32),
            jax.ShapeDtypeStruct((D, TOT), jnp.int32),
            jax.ShapeDtypeStruct((D, TOT), jnp.int32),
            jax.ShapeDtypeStruct((B, NN, C, C, C), jnp.float32),
        ],
    )(dps)
    return tuple(out)
